# baseline (speedup 1.0000x reference)
import functools

import jax
import jax.numpy as jnp
from jax import lax
from jax.experimental import pallas as pl
from jax.experimental.pallas import tpu as pltpu
from jax.experimental.pallas import tpu_sc as plsc

MAX_LEN = 200
BATCH = 4096
HIDDEN = 64
NUM_ROWS = MAX_LEN * BATCH

_NC = 2
_NS = 16
_NW = _NC * _NS
_B_PER_W = NUM_ROWS // _NW
_CHUNK = 512
_NCHUNKS = _B_PER_W // _CHUNK


def _make_gather():
    mesh = plsc.VectorSubcoreMesh(core_axis_name="c", subcore_axis_name="s")

    @functools.partial(
        pl.kernel,
        mesh=mesh,
        out_type=jax.ShapeDtypeStruct((NUM_ROWS, HIDDEN), jnp.float32),
        compiler_params=pltpu.CompilerParams(use_tc_tiling_on_sc=False),
        scratch_types=[
            pltpu.VMEM((_CHUNK,), jnp.int32),
            pltpu.VMEM((_CHUNK, HIDDEN), jnp.float32),
            pltpu.SemaphoreType.DMA,
        ],
    )
    def gather(table_hbm, idx_hbm, out_hbm, idx_v, rows_v, sem):
        wid = lax.axis_index("s") * _NC + lax.axis_index("c")
        base = wid * _B_PER_W

        def chunk_body(i, _):
            off = base + i * _CHUNK
            pltpu.sync_copy(idx_hbm.at[pl.ds(off, _CHUNK)], idx_v)
            pltpu.async_copy(table_hbm.at[idx_v], rows_v, sem).wait()
            pltpu.sync_copy(rows_v, out_hbm.at[pl.ds(off, _CHUNK)])
            return 0

        lax.fori_loop(0, _NCHUNKS, chunk_body, 0)

    return gather


_gather = _make_gather()


def kernel(event_activities, activity_embedding):
    idx_flat = event_activities.reshape(NUM_ROWS).astype(jnp.int32)
    out_flat = _gather(activity_embedding, idx_flat)
    return out_flat.reshape(MAX_LEN, BATCH, HIDDEN)


# speedup vs baseline: 1.0383x; 1.0383x over previous
import functools

import jax
import jax.numpy as jnp
from jax import lax
from jax.experimental import pallas as pl
from jax.experimental.pallas import tpu as pltpu
from jax.experimental.pallas import tpu_sc as plsc

MAX_LEN = 200
BATCH = 4096
HIDDEN = 64
NUM_ROWS = MAX_LEN * BATCH

_NC = 2
_NS = 16
_NW = _NC * _NS
_B_PER_W = NUM_ROWS // _NW
_CHUNK = 512
_NBUF = 2
_NCHUNKS = _B_PER_W // _CHUNK
_NGROUPS = _NCHUNKS // _NBUF


def _make_gather():
    mesh = plsc.VectorSubcoreMesh(core_axis_name="c", subcore_axis_name="s")

    @functools.partial(
        pl.kernel,
        mesh=mesh,
        out_type=jax.ShapeDtypeStruct((NUM_ROWS, HIDDEN), jnp.float32),
        compiler_params=pltpu.CompilerParams(use_tc_tiling_on_sc=False),
        scratch_types=[
            pltpu.VMEM((_B_PER_W,), jnp.int32),
            pltpu.VMEM((_NBUF, _CHUNK, HIDDEN), jnp.float32),
            pltpu.SemaphoreType.DMA((_NBUF,)),
            pltpu.SemaphoreType.DMA((_NBUF,)),
        ],
    )
    def gather(table_hbm, idx_hbm, out_hbm, idx_v, rows_v, gsem, wsem):
        wid = lax.axis_index("s") * _NC + lax.axis_index("c")
        base = wid * _B_PER_W
        pltpu.sync_copy(idx_hbm.at[pl.ds(base, _B_PER_W)], idx_v)

        def start_gather(i, b):
            pltpu.async_copy(
                table_hbm.at[idx_v.at[pl.ds(i * _CHUNK, _CHUNK)]],
                rows_v.at[b],
                gsem.at[b],
            )

        def wait_gather(b):
            pltpu.make_async_copy(
                table_hbm.at[idx_v.at[pl.ds(0, _CHUNK)]], rows_v.at[b], gsem.at[b]
            ).wait()

        def start_write(i, b):
            pltpu.async_copy(
                rows_v.at[b], out_hbm.at[pl.ds(base + i * _CHUNK, _CHUNK)], wsem.at[b]
            )

        def wait_write(b):
            pltpu.make_async_copy(
                rows_v.at[b], out_hbm.at[pl.ds(0, _CHUNK)], wsem.at[b]
            ).wait()

        for b in range(_NBUF):
            start_gather(b, b)

        def group(j, _):
            for b in range(_NBUF):
                wait_gather(b)
                start_write(j * _NBUF + b, b)
            for b in range(_NBUF):
                wait_write(b)
                start_gather((j + 1) * _NBUF + b, b)
            return 0

        lax.fori_loop(0, _NGROUPS - 1, group, 0)

        last = (_NGROUPS - 1) * _NBUF
        for b in range(_NBUF):
            wait_gather(b)
            start_write(last + b, b)
        for b in range(_NBUF):
            wait_write(b)

    return gather


_gather = _make_gather()


def kernel(event_activities, activity_embedding):
    idx_flat = event_activities.reshape(NUM_ROWS).astype(jnp.int32)
    out_flat = _gather(activity_embedding, idx_flat)
    return out_flat.reshape(MAX_LEN, BATCH, HIDDEN)
